# Initial kernel scaffold; baseline (speedup 1.0000x reference)
import jax
import jax.numpy as jnp
from jax.experimental import pallas as pl


def kernel(Xq, Xt, u_idx, v_idx, Waq, baq, Wat, bat, Wvq, bvq, Wvt, bvt, Wmq, bmq, Wmt, bmt):
    raise NotImplementedError("write your pallas kernel here")


# speedup vs baseline: 1.5851x; 1.5851x over previous
import jax
import jax.numpy as jnp
from jax.experimental import pallas as pl
from jax.experimental.pallas import tpu as pltpu

NQ = 10000
NT = 10000
E = 160000
D = 256
BM = 1000


def _elu(x):
    return jnp.where(x > 0, x, jnp.exp(jnp.minimum(x, 0.0)) - 1.0)


def _affine_elu_body(x_ref, w_ref, b_ref, o_ref):
    acc = jnp.dot(x_ref[...], w_ref[...], preferred_element_type=jnp.float32)
    o_ref[...] = _elu(acc + b_ref[...])


def _affine_elu(x, w, b):
    n = x.shape[0]
    return pl.pallas_call(
        _affine_elu_body,
        grid=(n // BM,),
        in_specs=[
            pl.BlockSpec((BM, D), lambda i: (i, 0)),
            pl.BlockSpec((D, D), lambda i: (0, 0)),
            pl.BlockSpec((1, D), lambda i: (0, 0)),
        ],
        out_specs=pl.BlockSpec((BM, D), lambda i: (i, 0)),
        out_shape=jax.ShapeDtypeStruct((n, D), jnp.float32),
    )(x, w, b.reshape(1, D))


def _merge2_body(x_ref, t_ref, w_ref, b_ref, o_ref):
    acc = jnp.dot(x_ref[...], w_ref[:D, :], preferred_element_type=jnp.float32)
    acc += jnp.dot(t_ref[...], w_ref[D:, :], preferred_element_type=jnp.float32)
    o_ref[...] = acc + b_ref[...]


def _merge2(x, t, w, brow):
    n = x.shape[0]
    return pl.pallas_call(
        _merge2_body,
        grid=(n // BM,),
        in_specs=[
            pl.BlockSpec((BM, D), lambda i: (i, 0)),
            pl.BlockSpec((BM, D), lambda i: (i, 0)),
            pl.BlockSpec((2 * D, D), lambda i: (0, 0)),
            pl.BlockSpec((1, D), lambda i: (0, 0)),
        ],
        out_specs=pl.BlockSpec((BM, D), lambda i: (i, 0)),
        out_shape=jax.ShapeDtypeStruct((n, D), jnp.float32),
    )(x, t, w, brow.reshape(1, D))


def kernel(Xq, Xt, u_idx, v_idx, Waq, baq, Wat, bat, Wvq, bvq, Wvt, bvt, Wmq, bmq, Wmt, bmt):
    Aq = _affine_elu(Xq, Waq, baq)
    At = _affine_elu(Xt, Wat, bat)
    Vt = _affine_elu(Xt, Wvt, bvt)
    Vq = _affine_elu(Xq, Wvq, bvq)

    logits = jnp.sum(Aq[u_idx] * At[v_idx], axis=1)

    has_u = jnp.zeros((NQ,), jnp.float32).at[u_idx].max(1.0)
    Xt2q = Vt * has_u[:, None]

    m_v = jax.ops.segment_max(logits, v_idx, num_segments=NT)
    ex = jnp.exp(logits - m_v[v_idx])
    s_v = jax.ops.segment_sum(ex, v_idx, num_segments=NT)
    w_e = ex / s_v[v_idx]
    Xq2t = jax.ops.segment_sum(w_e[:, None] * Vq[u_idx], v_idx, num_segments=NT)

    Xq_merged = _merge2(Xq, Xt2q, Wmq, bmq)
    qrow = jnp.mean(Xq, axis=0) @ Wmt[2 * D :, :] + bmt
    Xt_merged = _merge2(Xt, Xq2t, Wmt[: 2 * D, :], qrow)
    return (Xq_merged, Xt_merged)


# speedup vs baseline: 7.4739x; 4.7151x over previous
import jax
import jax.numpy as jnp
from jax import lax
from jax.experimental import pallas as pl
from jax.experimental.pallas import tpu as pltpu
from jax.experimental.pallas import tpu_sc as plsc

NQ = 10000
NT = 10000
E = 160000
D = 256
BM = 1000

SC_NC = 2
SC_NS = 16
NW = SC_NC * SC_NS
CHUNK = 128
NCHUNKS = E // CHUNK
NGROUPS = CHUNK // 16

_MESH = dict(core_axis_name="c", subcore_axis_name="s", num_cores=SC_NC,
             num_subcores=SC_NS)


def _elu(x):
    return jnp.where(x > 0, x, jnp.exp(jnp.minimum(x, 0.0)) - 1.0)


def _affine_elu_body(x_ref, w_ref, b_ref, o_ref):
    acc = jnp.dot(x_ref[...], w_ref[...], preferred_element_type=jnp.float32)
    o_ref[...] = _elu(acc + b_ref[...])


def _affine_elu(x, w, b):
    n = x.shape[0]
    return pl.pallas_call(
        _affine_elu_body,
        grid=(n // BM,),
        in_specs=[
            pl.BlockSpec((BM, D), lambda i: (i, 0)),
            pl.BlockSpec((D, D), lambda i: (0, 0)),
            pl.BlockSpec((1, D), lambda i: (0, 0)),
        ],
        out_specs=pl.BlockSpec((BM, D), lambda i: (i, 0)),
        out_shape=jax.ShapeDtypeStruct((n, D), jnp.float32),
    )(x, w, b.reshape(1, D))


def _affine_elu_split_body(x_ref, w_ref, b_ref, o_ref):
    h = pl.program_id(1)
    acc = jnp.dot(x_ref[...], w_ref[:, pl.ds(h * 128, 128)],
                  preferred_element_type=jnp.float32)
    o_ref[...] = _elu(acc + b_ref[:, pl.ds(h * 128, 128)])[None]


def _affine_elu_split(x, w, b):
    n = x.shape[0]
    return pl.pallas_call(
        _affine_elu_split_body,
        grid=(n // BM, 2),
        in_specs=[
            pl.BlockSpec((BM, D), lambda i, h: (i, 0)),
            pl.BlockSpec((D, D), lambda i, h: (0, 0)),
            pl.BlockSpec((1, D), lambda i, h: (0, 0)),
        ],
        out_specs=pl.BlockSpec((1, BM, 128), lambda i, h: (h, i, 0)),
        out_shape=jax.ShapeDtypeStruct((2, n, 128), jnp.float32),
    )(x, w, b.reshape(1, D))


def _merge2_body(x_ref, t_ref, w_ref, b_ref, o_ref):
    acc = jnp.dot(x_ref[...], w_ref[:D, :], preferred_element_type=jnp.float32)
    acc += jnp.dot(t_ref[...], w_ref[D:, :], preferred_element_type=jnp.float32)
    o_ref[...] = acc + b_ref[...]


def _merge2(x, t, w, brow):
    n = x.shape[0]
    return pl.pallas_call(
        _merge2_body,
        grid=(n // BM,),
        in_specs=[
            pl.BlockSpec((BM, D), lambda i: (i, 0)),
            pl.BlockSpec((BM, D), lambda i: (i, 0)),
            pl.BlockSpec((2 * D, D), lambda i: (0, 0)),
            pl.BlockSpec((1, D), lambda i: (0, 0)),
        ],
        out_specs=pl.BlockSpec((BM, D), lambda i: (i, 0)),
        out_shape=jax.ShapeDtypeStruct((n, D), jnp.float32),
    )(x, t, w, brow.reshape(1, D))


def _merge_h_body(x_ref, h0_ref, h1_ref, w_ref, b_ref, o_ref):
    acc = jnp.dot(x_ref[...], w_ref[:D, :], preferred_element_type=jnp.float32)
    acc += jnp.dot(h0_ref[...], w_ref[D:D + 128, :],
                   preferred_element_type=jnp.float32)
    acc += jnp.dot(h1_ref[...], w_ref[D + 128:, :],
                   preferred_element_type=jnp.float32)
    o_ref[...] = acc + b_ref[...]


def _merge_h(x, halves, w, brow):
    n = x.shape[0]
    return pl.pallas_call(
        _merge_h_body,
        grid=(n // BM,),
        in_specs=[
            pl.BlockSpec((BM, D), lambda i: (i, 0)),
            pl.BlockSpec((BM, 128), lambda i: (i, 0)),
            pl.BlockSpec((BM, 128), lambda i: (i + n // BM, 0)),
            pl.BlockSpec((2 * D, D), lambda i: (0, 0)),
            pl.BlockSpec((1, D), lambda i: (0, 0)),
        ],
        out_specs=pl.BlockSpec((BM, D), lambda i: (i, 0)),
        out_shape=jax.ShapeDtypeStruct((n, D), jnp.float32),
    )(x, halves, halves, w, brow.reshape(1, D))


def _colreduce_body_max(x_ref, o_ref):
    o_ref[...] = jnp.max(x_ref[...], axis=0, keepdims=True)


def _colreduce_body_sum(x_ref, o_ref):
    o_ref[...] = jnp.sum(x_ref[...], axis=0, keepdims=True)


def _colreduce(x, kind):
    rows, n = x.shape
    body = _colreduce_body_max if kind == "max" else _colreduce_body_sum
    out = pl.pallas_call(
        body,
        grid=(1,),
        in_specs=[pl.BlockSpec((rows, n), lambda i: (0, 0))],
        out_specs=pl.BlockSpec((1, n), lambda i: (0, 0)),
        out_shape=jax.ShapeDtypeStruct((1, n), jnp.float32),
    )(x)
    return out.reshape(n)


def _wid(c, s):
    return s * SC_NC + c


def _nchunks_for(wid, stride):
    return (NCHUNKS - wid + stride - 1) // stride


def _sc_logits_body(aq_h, at_h, u_h, v_h, lg_h, mp_h, up_h,
                    u_v, v_v, aq_v, at_v, lg_v, tr_v, m_loc, um_loc, sem):
    c = lax.axis_index("c")
    s = lax.axis_index("s")
    wid = _wid(c, s)
    iota = lax.iota(jnp.int32, 16)
    neg = jnp.full((16,), -1e30, jnp.float32)
    one = jnp.ones((16,), jnp.float32)
    zero = jnp.zeros((16,), jnp.float32)

    def init_body(i, carry):
        m_loc[pl.ds(i * 16, 16)] = neg
        um_loc[pl.ds(i * 16, 16)] = zero
        return carry

    lax.fori_loop(0, NT // 16, init_body, 0)

    def chunk_body(ci, carry):
        base = (ci * NW + wid) * CHUNK
        pltpu.sync_copy(u_h.at[pl.ds(base, CHUNK)], u_v)
        pltpu.sync_copy(v_h.at[pl.ds(base, CHUNK)], v_v)
        pltpu.async_copy(aq_h.at[u_v], aq_v, sem).wait()
        pltpu.async_copy(at_h.at[v_v], at_v, sem).wait()

        def group_body(g, carry2):
            gb = g * 16
            for j in range(16):
                acc = aq_v[gb + j, pl.ds(0, 16)] * at_v[gb + j, pl.ds(0, 16)]
                for k in range(1, 16):
                    acc = acc + (aq_v[gb + j, pl.ds(k * 16, 16)]
                                 * at_v[gb + j, pl.ds(k * 16, 16)])
                tr_v[pl.ds(j * 17, 16)] = acc
            lg = plsc.load_gather(tr_v, [iota * 17])
            for k in range(1, 16):
                lg = lg + plsc.load_gather(tr_v, [iota * 17 + k])
            lg_v[pl.ds(gb, 16)] = lg
            vseg = v_v[pl.ds(gb, 16)]
            useg = u_v[pl.ds(gb, 16)]
            plsc.store_scatter(um_loc, [useg], one)

            def mx_once(_):
                cur = plsc.load_gather(m_loc, [vseg])
                need = lg > cur
                plsc.store_scatter(m_loc, [vseg], jnp.maximum(cur, lg),
                                   mask=need)
                cur2 = plsc.load_gather(m_loc, [vseg])
                return jnp.any(lg > cur2)

            lax.while_loop(lambda p: p, mx_once, mx_once(True))
            return carry2

        lax.fori_loop(0, NGROUPS, group_body, 0)
        pltpu.sync_copy(lg_v, lg_h.at[pl.ds(base, CHUNK)])
        return carry

    lax.fori_loop(0, _nchunks_for(wid, NW), chunk_body, 0)
    pltpu.sync_copy(m_loc, mp_h.at[wid])
    pltpu.sync_copy(um_loc, up_h.at[wid])


def _sc_logits(Aq, At, u_idx, v_idx):
    f = pl.kernel(
        _sc_logits_body,
        out_type=[
            jax.ShapeDtypeStruct((E,), jnp.float32),
            jax.ShapeDtypeStruct((NW, NT), jnp.float32),
            jax.ShapeDtypeStruct((NW, NQ), jnp.float32),
        ],
        mesh=plsc.VectorSubcoreMesh(**_MESH),
        compiler_params=pltpu.CompilerParams(needs_layout_passes=False),
        scratch_types=[
            pltpu.VMEM((CHUNK,), jnp.int32),
            pltpu.VMEM((CHUNK,), jnp.int32),
            pltpu.VMEM((CHUNK, D), jnp.float32),
            pltpu.VMEM((CHUNK, D), jnp.float32),
            pltpu.VMEM((CHUNK,), jnp.float32),
            pltpu.VMEM((16 * 17,), jnp.float32),
            pltpu.VMEM((NT,), jnp.float32),
            pltpu.VMEM((NQ,), jnp.float32),
            pltpu.SemaphoreType.DMA,
        ],
    )
    return f(Aq, At, u_idx, v_idx)


def _sc_exp_body(lg_h, v_h, mv_h, ex_h, sp_h,
                 v_v, lg_v, ex_v, mv_t, s_loc, tag, sem):
    c = lax.axis_index("c")
    s = lax.axis_index("s")
    wid = _wid(c, s)
    iota = lax.iota(jnp.int32, 16)
    zero = jnp.zeros((16,), jnp.float32)

    pltpu.sync_copy(mv_h, mv_t)

    def init_body(i, carry):
        s_loc[pl.ds(i * 16, 16)] = zero
        return carry

    lax.fori_loop(0, NT // 16, init_body, 0)

    def chunk_body(ci, carry):
        base = (ci * NW + wid) * CHUNK
        pltpu.sync_copy(v_h.at[pl.ds(base, CHUNK)], v_v)
        pltpu.sync_copy(lg_h.at[pl.ds(base, CHUNK)], lg_v)

        def group_body(g, carry2):
            gb = g * 16
            vseg = v_v[pl.ds(gb, 16)]
            m = plsc.load_gather(mv_t, [vseg])
            ex = jnp.exp(lg_v[pl.ds(gb, 16)] - m)
            ex_v[pl.ds(gb, 16)] = ex

            def claim(st):
                plsc.store_scatter(tag, [vseg], iota, mask=st)
                t = plsc.load_gather(tag, [vseg])
                win = st & (t == iota)
                cur = plsc.load_gather(s_loc, [vseg])
                plsc.store_scatter(s_loc, [vseg], cur + ex, mask=win)
                return st & jnp.logical_not(win)

            st0 = iota == iota
            lax.while_loop(lambda st: jnp.any(st), claim, claim(st0))
            return carry2

        lax.fori_loop(0, NGROUPS, group_body, 0)
        pltpu.sync_copy(ex_v, ex_h.at[pl.ds(base, CHUNK)])
        return carry

    lax.fori_loop(0, _nchunks_for(wid, NW), chunk_body, 0)
    pltpu.sync_copy(s_loc, sp_h.at[wid])


def _sc_exp(logits, v_idx, m_v):
    f = pl.kernel(
        _sc_exp_body,
        out_type=[
            jax.ShapeDtypeStruct((E,), jnp.float32),
            jax.ShapeDtypeStruct((NW, NT), jnp.float32),
        ],
        mesh=plsc.VectorSubcoreMesh(**_MESH),
        compiler_params=pltpu.CompilerParams(needs_layout_passes=False),
        scratch_types=[
            pltpu.VMEM((CHUNK,), jnp.int32),
            pltpu.VMEM((CHUNK,), jnp.float32),
            pltpu.VMEM((CHUNK,), jnp.float32),
            pltpu.VMEM((NT,), jnp.float32),
            pltpu.VMEM((NT,), jnp.float32),
            pltpu.VMEM((NT,), jnp.int32),
            pltpu.SemaphoreType.DMA,
        ],
    )
    return f(logits, v_idx, m_v)


def _sc_scatter_body(ex_h, u_h, v_h, sv_h, vqh_h, zero_h, o_h,
                     u_v, v_v, ex_v, w_v, rows_v, sv_t, acc_sh, sem):
    c = lax.axis_index("c")
    s = lax.axis_index("s")
    rows_per = 624
    uoff = c * NQ

    pltpu.sync_copy(sv_h, sv_t)
    pltpu.sync_copy(zero_h.at[pl.ds(s * rows_per, rows_per)],
                    acc_sh.at[pl.ds(s * rows_per, rows_per)])

    @pl.when(s == SC_NS - 1)
    def _zero_tail():
        pltpu.sync_copy(zero_h.at[pl.ds(SC_NS * rows_per, NT - SC_NS * rows_per)],
                        acc_sh.at[pl.ds(SC_NS * rows_per, NT - SC_NS * rows_per)])

    plsc.subcore_barrier()

    def chunk_body(ci, carry):
        base = (ci * SC_NS + s) * CHUNK
        pltpu.sync_copy(u_h.at[pl.ds(base, CHUNK)], u_v)
        pltpu.sync_copy(v_h.at[pl.ds(base, CHUNK)], v_v)
        pltpu.sync_copy(ex_h.at[pl.ds(base, CHUNK)], ex_v)

        def group_body(g, carry2):
            gb = g * 16
            vseg = v_v[pl.ds(gb, 16)]
            sv = plsc.load_gather(sv_t, [vseg])
            w_v[pl.ds(gb, 16)] = ex_v[pl.ds(gb, 16)] / sv
            u_v[pl.ds(gb, 16)] = u_v[pl.ds(gb, 16)] + uoff
            return carry2

        lax.fori_loop(0, NGROUPS, group_body, 0)
        pltpu.async_copy(vqh_h.at[u_v], rows_v, sem).wait()

        def row_body(j, carry2):
            wrow = plsc.load_gather(w_v, [jnp.full((16,), j, jnp.int32)])
            for k in range(8):
                rows_v[j, pl.ds(k * 16, 16)] = (
                    rows_v[j, pl.ds(k * 16, 16)] * wrow)
            return carry2

        lax.fori_loop(0, CHUNK, row_body, 0, unroll=4)
        pltpu.sync_copy(rows_v, acc_sh.at[v_v], add=True)
        return carry

    lax.fori_loop(0, (NCHUNKS - s + SC_NS - 1) // SC_NS, chunk_body, 0)
    plsc.subcore_barrier()
    pltpu.sync_copy(acc_sh.at[pl.ds(s * rows_per, rows_per)],
                    o_h.at[pl.ds(c * NT + s * rows_per, rows_per)])

    @pl.when(s == SC_NS - 1)
    def _out_tail():
        pltpu.sync_copy(acc_sh.at[pl.ds(SC_NS * rows_per, NT - SC_NS * rows_per)],
                        o_h.at[pl.ds(c * NT + SC_NS * rows_per, NT - SC_NS * rows_per)])


def _sc_scatter(ex, u_idx, v_idx, s_v, VqH, zeros):
    f = pl.kernel(
        _sc_scatter_body,
        out_type=[jax.ShapeDtypeStruct((SC_NC * NT, 128), jnp.float32)],
        mesh=plsc.VectorSubcoreMesh(**_MESH),
        compiler_params=pltpu.CompilerParams(needs_layout_passes=False),
        scratch_types=[
            pltpu.VMEM((CHUNK,), jnp.int32),
            pltpu.VMEM((CHUNK,), jnp.int32),
            pltpu.VMEM((CHUNK,), jnp.float32),
            pltpu.VMEM((CHUNK,), jnp.float32),
            pltpu.VMEM((CHUNK, 128), jnp.float32),
            pltpu.VMEM((NT,), jnp.float32),
            pltpu.VMEM_SHARED((NT, 128), jnp.float32),
            pltpu.SemaphoreType.DMA,
        ],
    )
    return f(ex, u_idx, v_idx, s_v, VqH, zeros)[0]


def kernel(Xq, Xt, u_idx, v_idx, Waq, baq, Wat, bat, Wvq, bvq, Wvt, bvt, Wmq, bmq, Wmt, bmt):
    Aq = _affine_elu(Xq, Waq, baq)
    At = _affine_elu(Xt, Wat, bat)
    Vt = _affine_elu(Xt, Wvt, bvt)
    VqH = _affine_elu_split(Xq, Wvq, bvq).reshape(2 * NQ, 128)

    logits, m_part, u_part = _sc_logits(Aq, At, u_idx, v_idx)
    m_v = _colreduce(m_part, "max")
    has_u = _colreduce(u_part, "max")

    ex, s_part = _sc_exp(logits, v_idx, m_v)
    s_v = _colreduce(s_part, "sum")

    zeros = jnp.zeros((NT, 128), jnp.float32)
    P = _sc_scatter(ex, u_idx, v_idx, s_v, VqH, zeros)

    Xt2q = Vt * has_u[:, None]
    Xq_merged = _merge2(Xq, Xt2q, Wmq, bmq)
    qrow = jnp.mean(Xq, axis=0) @ Wmt[2 * D:, :] + bmt
    Xt_merged = _merge_h(Xt, P, Wmt[:2 * D, :], qrow)
    return (Xq_merged, Xt_merged)


# speedup vs baseline: 12.4743x; 1.6690x over previous
import jax
import jax.numpy as jnp
from jax import lax
from jax.experimental import pallas as pl
from jax.experimental.pallas import tpu as pltpu
from jax.experimental.pallas import tpu_sc as plsc

NQ = 10000
NT = 10000
E = 160000
D = 256
BM = 1000

SC_NC = 2
SC_NS = 16
NW = SC_NC * SC_NS
CHUNK = 128
NCHUNKS = E // CHUNK
NGROUPS = CHUNK // 16

_MESH = dict(core_axis_name="c", subcore_axis_name="s", num_cores=SC_NC,
             num_subcores=SC_NS)


def _elu(x):
    return jnp.where(x > 0, x, jnp.exp(jnp.minimum(x, 0.0)) - 1.0)


def _affine_elu_body(x_ref, w_ref, b_ref, o_ref):
    acc = jnp.dot(x_ref[...], w_ref[...], preferred_element_type=jnp.float32)
    o_ref[...] = _elu(acc + b_ref[...])


def _affine_elu(x, w, b):
    n = x.shape[0]
    return pl.pallas_call(
        _affine_elu_body,
        grid=(n // BM,),
        in_specs=[
            pl.BlockSpec((BM, D), lambda i: (i, 0)),
            pl.BlockSpec((D, D), lambda i: (0, 0)),
            pl.BlockSpec((1, D), lambda i: (0, 0)),
        ],
        out_specs=pl.BlockSpec((BM, D), lambda i: (i, 0)),
        out_shape=jax.ShapeDtypeStruct((n, D), jnp.float32),
    )(x, w, b.reshape(1, D))


def _affine_elu_split_body(x_ref, w_ref, b_ref, o_ref):
    h = pl.program_id(1)
    acc = jnp.dot(x_ref[...], w_ref[:, pl.ds(h * 128, 128)],
                  preferred_element_type=jnp.float32)
    o_ref[...] = _elu(acc + b_ref[:, pl.ds(h * 128, 128)])[None]


def _affine_elu_split(x, w, b):
    n = x.shape[0]
    return pl.pallas_call(
        _affine_elu_split_body,
        grid=(n // BM, 2),
        in_specs=[
            pl.BlockSpec((BM, D), lambda i, h: (i, 0)),
            pl.BlockSpec((D, D), lambda i, h: (0, 0)),
            pl.BlockSpec((1, D), lambda i, h: (0, 0)),
        ],
        out_specs=pl.BlockSpec((1, BM, 128), lambda i, h: (h, i, 0)),
        out_shape=jax.ShapeDtypeStruct((2, n, 128), jnp.float32),
    )(x, w, b.reshape(1, D))


def _merge2_body(x_ref, t_ref, w_ref, b_ref, o_ref):
    acc = jnp.dot(x_ref[...], w_ref[:D, :], preferred_element_type=jnp.float32)
    acc += jnp.dot(t_ref[...], w_ref[D:, :], preferred_element_type=jnp.float32)
    o_ref[...] = acc + b_ref[...]


def _merge2(x, t, w, brow):
    n = x.shape[0]
    return pl.pallas_call(
        _merge2_body,
        grid=(n // BM,),
        in_specs=[
            pl.BlockSpec((BM, D), lambda i: (i, 0)),
            pl.BlockSpec((BM, D), lambda i: (i, 0)),
            pl.BlockSpec((2 * D, D), lambda i: (0, 0)),
            pl.BlockSpec((1, D), lambda i: (0, 0)),
        ],
        out_specs=pl.BlockSpec((BM, D), lambda i: (i, 0)),
        out_shape=jax.ShapeDtypeStruct((n, D), jnp.float32),
    )(x, t, w, brow.reshape(1, D))


def _merge_h_body(x_ref, h0_ref, h1_ref, w_ref, b_ref, o_ref):
    acc = jnp.dot(x_ref[...], w_ref[:D, :], preferred_element_type=jnp.float32)
    acc += jnp.dot(h0_ref[...], w_ref[D:D + 128, :],
                   preferred_element_type=jnp.float32)
    acc += jnp.dot(h1_ref[...], w_ref[D + 128:, :],
                   preferred_element_type=jnp.float32)
    o_ref[...] = acc + b_ref[...]


def _merge_h(x, halves, w, brow):
    n = x.shape[0]
    return pl.pallas_call(
        _merge_h_body,
        grid=(n // BM,),
        in_specs=[
            pl.BlockSpec((BM, D), lambda i: (i, 0)),
            pl.BlockSpec((BM, 128), lambda i: (i, 0)),
            pl.BlockSpec((BM, 128), lambda i: (i + n // BM, 0)),
            pl.BlockSpec((2 * D, D), lambda i: (0, 0)),
            pl.BlockSpec((1, D), lambda i: (0, 0)),
        ],
        out_specs=pl.BlockSpec((BM, D), lambda i: (i, 0)),
        out_shape=jax.ShapeDtypeStruct((n, D), jnp.float32),
    )(x, halves, halves, w, brow.reshape(1, D))


def _colreduce_body_max(x_ref, o_ref):
    o_ref[...] = jnp.max(x_ref[...], axis=0, keepdims=True)


def _colreduce_body_sum(x_ref, o_ref):
    o_ref[...] = jnp.sum(x_ref[...], axis=0, keepdims=True)


def _colreduce(x, kind):
    rows, n = x.shape
    body = _colreduce_body_max if kind == "max" else _colreduce_body_sum
    out = pl.pallas_call(
        body,
        grid=(1,),
        in_specs=[pl.BlockSpec((rows, n), lambda i: (0, 0))],
        out_specs=pl.BlockSpec((1, n), lambda i: (0, 0)),
        out_shape=jax.ShapeDtypeStruct((1, n), jnp.float32),
    )(x)
    return out.reshape(n)


def _wid(c, s):
    return s * SC_NC + c


def _nchunks_for(wid, stride):
    return (NCHUNKS - wid + stride - 1) // stride


K1C = 64
K1NCH = E // K1C
K1BASE = K1NCH // NW
K1SPAN = (K1BASE + 1) * K1C
EPAD = NW * K1SPAN


def _sc_logits_body(aq_h, at_h, u_h, v_h, lg_h, mp_h, up_h,
                    u_all, v_all, aq0, at0, aq1, at1, lg_all, tr_v,
                    m_loc, um_loc, sa0, sb0, sa1, sb1):
    c = lax.axis_index("c")
    s = lax.axis_index("s")
    wid = _wid(c, s)
    iota = lax.iota(jnp.int32, 16)
    neg = jnp.full((16,), -1e30, jnp.float32)
    one = jnp.ones((16,), jnp.float32)
    zero = jnp.zeros((16,), jnp.float32)

    def init_body(i, carry):
        m_loc[pl.ds(i * 16, 16)] = neg
        um_loc[pl.ds(i * 16, 16)] = zero
        return carry

    lax.fori_loop(0, NT // 16, init_body, 0)

    nck = K1BASE + jnp.where(wid < 4, 1, 0)
    start_chunk = wid * K1BASE + jnp.minimum(wid, 4)
    base_e = start_chunk * K1C
    pltpu.sync_copy(u_h.at[pl.ds(base_e, K1SPAN)], u_all)
    pltpu.sync_copy(v_h.at[pl.ds(base_e, K1SPAN)], v_all)

    def issue(ci, aqb, atb, sa, sb):
        ub = u_all.at[pl.ds(ci * K1C, K1C)]
        vb = v_all.at[pl.ds(ci * K1C, K1C)]
        pltpu.async_copy(aq_h.at[ub], aqb, sa)
        pltpu.async_copy(at_h.at[vb], atb, sb)

    def wait(ci, aqb, atb, sa, sb):
        ub = u_all.at[pl.ds(ci * K1C, K1C)]
        vb = v_all.at[pl.ds(ci * K1C, K1C)]
        pltpu.make_async_copy(aq_h.at[ub], aqb, sa).wait()
        pltpu.make_async_copy(at_h.at[vb], atb, sb).wait()

    def process(ci, aq_v, at_v):
        def group_body(g, carry2):
            gb = g * 16
            eb = ci * K1C + gb
            for j in range(16):
                acc = aq_v[gb + j, pl.ds(0, 16)] * at_v[gb + j, pl.ds(0, 16)]
                for k in range(1, 16):
                    acc = acc + (aq_v[gb + j, pl.ds(k * 16, 16)]
                                 * at_v[gb + j, pl.ds(k * 16, 16)])
                tr_v[pl.ds(j * 17, 16)] = acc
            lg = plsc.load_gather(tr_v, [iota * 17])
            for k in range(1, 16):
                lg = lg + plsc.load_gather(tr_v, [iota * 17 + k])
            lg_all[pl.ds(eb, 16)] = lg
            vseg = v_all[pl.ds(eb, 16)]
            useg = u_all[pl.ds(eb, 16)]
            plsc.store_scatter(um_loc, [useg], one)

            def mx_once(_):
                cur = plsc.load_gather(m_loc, [vseg])
                need = lg > cur
                plsc.store_scatter(m_loc, [vseg], jnp.maximum(cur, lg),
                                   mask=need)
                cur2 = plsc.load_gather(m_loc, [vseg])
                return jnp.any(lg > cur2)

            lax.while_loop(lambda p: p, mx_once, mx_once(True))
            return carry2

        lax.fori_loop(0, K1C // 16, group_body, 0)

    issue(0, aq0, at0, sa0, sb0)

    def pair_body(p, carry):
        c0 = 2 * p
        wait(c0, aq0, at0, sa0, sb0)
        issue(c0 + 1, aq1, at1, sa1, sb1)
        process(c0, aq0, at0)
        wait(c0 + 1, aq1, at1, sa1, sb1)

        @pl.when(c0 + 2 < nck)
        def _():
            issue(c0 + 2, aq0, at0, sa0, sb0)

        process(c0 + 1, aq1, at1)
        return carry

    lax.fori_loop(0, K1BASE // 2, pair_body, 0)

    @pl.when(nck == K1BASE + 1)
    def _last():
        wait(K1BASE, aq0, at0, sa0, sb0)
        process(K1BASE, aq0, at0)

    n_main = K1BASE * K1C
    pltpu.sync_copy(lg_all.at[pl.ds(0, n_main)], lg_h.at[pl.ds(base_e, n_main)])

    @pl.when(nck == K1BASE + 1)
    def _tail():
        pltpu.sync_copy(lg_all.at[pl.ds(n_main, K1C)],
                        lg_h.at[pl.ds(base_e + n_main, K1C)])

    pltpu.sync_copy(m_loc, mp_h.at[wid])
    pltpu.sync_copy(um_loc, up_h.at[wid])


def _sc_logits(Aq, At, u_pad, v_pad):
    f = pl.kernel(
        _sc_logits_body,
        out_type=[
            jax.ShapeDtypeStruct((E,), jnp.float32),
            jax.ShapeDtypeStruct((NW, NT), jnp.float32),
            jax.ShapeDtypeStruct((NW, NQ), jnp.float32),
        ],
        mesh=plsc.VectorSubcoreMesh(**_MESH),
        compiler_params=pltpu.CompilerParams(needs_layout_passes=False),
        scratch_types=[
            pltpu.VMEM((K1SPAN,), jnp.int32),
            pltpu.VMEM((K1SPAN,), jnp.int32),
            pltpu.VMEM((K1C, D), jnp.float32),
            pltpu.VMEM((K1C, D), jnp.float32),
            pltpu.VMEM((K1C, D), jnp.float32),
            pltpu.VMEM((K1C, D), jnp.float32),
            pltpu.VMEM((K1SPAN,), jnp.float32),
            pltpu.VMEM((16 * 17,), jnp.float32),
            pltpu.VMEM((NT,), jnp.float32),
            pltpu.VMEM((NQ,), jnp.float32),
            pltpu.SemaphoreType.DMA,
            pltpu.SemaphoreType.DMA,
            pltpu.SemaphoreType.DMA,
            pltpu.SemaphoreType.DMA,
        ],
    )
    return f(Aq, At, u_pad, v_pad)


def _sc_exp_body(lg_h, v_h, mv_h, ex_h, sp_h,
                 v_v, lg_v, ex_v, mv_t, s_loc, tag, sem):
    c = lax.axis_index("c")
    s = lax.axis_index("s")
    wid = _wid(c, s)
    iota = lax.iota(jnp.int32, 16)
    zero = jnp.zeros((16,), jnp.float32)

    pltpu.sync_copy(mv_h, mv_t)

    def init_body(i, carry):
        s_loc[pl.ds(i * 16, 16)] = zero
        return carry

    lax.fori_loop(0, NT // 16, init_body, 0)

    def chunk_body(ci, carry):
        base = (ci * NW + wid) * CHUNK
        pltpu.sync_copy(v_h.at[pl.ds(base, CHUNK)], v_v)
        pltpu.sync_copy(lg_h.at[pl.ds(base, CHUNK)], lg_v)

        def group_body(g, carry2):
            gb = g * 16
            vseg = v_v[pl.ds(gb, 16)]
            m = plsc.load_gather(mv_t, [vseg])
            ex = jnp.exp(lg_v[pl.ds(gb, 16)] - m)
            ex_v[pl.ds(gb, 16)] = ex

            def claim(st):
                plsc.store_scatter(tag, [vseg], iota, mask=st)
                t = plsc.load_gather(tag, [vseg])
                win = st & (t == iota)
                cur = plsc.load_gather(s_loc, [vseg])
                plsc.store_scatter(s_loc, [vseg], cur + ex, mask=win)
                return st & jnp.logical_not(win)

            st0 = iota == iota
            lax.while_loop(lambda st: jnp.any(st), claim, claim(st0))
            return carry2

        lax.fori_loop(0, NGROUPS, group_body, 0)
        pltpu.sync_copy(ex_v, ex_h.at[pl.ds(base, CHUNK)])
        return carry

    lax.fori_loop(0, _nchunks_for(wid, NW), chunk_body, 0)
    pltpu.sync_copy(s_loc, sp_h.at[wid])


def _sc_exp(logits, v_idx, m_v):
    f = pl.kernel(
        _sc_exp_body,
        out_type=[
            jax.ShapeDtypeStruct((EPAD,), jnp.float32),
            jax.ShapeDtypeStruct((NW, NT), jnp.float32),
        ],
        mesh=plsc.VectorSubcoreMesh(**_MESH),
        compiler_params=pltpu.CompilerParams(needs_layout_passes=False),
        scratch_types=[
            pltpu.VMEM((CHUNK,), jnp.int32),
            pltpu.VMEM((CHUNK,), jnp.float32),
            pltpu.VMEM((CHUNK,), jnp.float32),
            pltpu.VMEM((NT,), jnp.float32),
            pltpu.VMEM((NT,), jnp.float32),
            pltpu.VMEM((NT,), jnp.int32),
            pltpu.SemaphoreType.DMA,
        ],
    )
    return f(logits, v_idx, m_v)


K3BASE = K1NCH // SC_NS
K3SEG = (K1BASE + 2) // 2 * 2
K3SPAN = K3SEG * K1C


def _sc_scatter_body(ex_h, u_h, v_h, sv_h, vqh_h, zero_h, o_h,
                     u_all, v_all, w_all, v_stage, rows0, rows1,
                     sv_t, acc_sh, sem0, sem1):
    c = lax.axis_index("c")
    s = lax.axis_index("s")
    rows_per = 624
    uoff = c * NQ

    pltpu.sync_copy(sv_h, sv_t)
    pltpu.sync_copy(zero_h.at[pl.ds(s * rows_per, rows_per)],
                    acc_sh.at[pl.ds(s * rows_per, rows_per)])

    @pl.when(s == SC_NS - 1)
    def _zero_tail():
        pltpu.sync_copy(zero_h.at[pl.ds(SC_NS * rows_per, NT - SC_NS * rows_per)],
                        acc_sh.at[pl.ds(SC_NS * rows_per, NT - SC_NS * rows_per)])

    nck = K3BASE + jnp.where(s < 4, 1, 0)
    start_chunk = s * K3BASE + jnp.minimum(s, 4)
    plsc.subcore_barrier()

    def issue(ci, rowsb, sem):
        ub = u_all.at[pl.ds(ci * K1C, K1C)]
        pltpu.async_copy(vqh_h.at[ub], rowsb, sem)

    def wait(ci, rowsb, sem):
        ub = u_all.at[pl.ds(ci * K1C, K1C)]
        pltpu.make_async_copy(vqh_h.at[ub], rowsb, sem).wait()

    def process(ci, rowsb):
        def row_body(j, carry2):
            wrow = plsc.load_gather(
                w_all, [jnp.full((16,), ci * K1C + j, jnp.int32)])
            for k in range(8):
                rowsb[j, pl.ds(k * 16, 16)] = (
                    rowsb[j, pl.ds(k * 16, 16)] * wrow)
            return carry2

        lax.fori_loop(0, K1C, row_body, 0, unroll=4)
        for t in range(K1C // 16):
            v_stage[pl.ds(t * 16, 16)] = v_all[pl.ds(ci * K1C + t * 16, 16)]
        pltpu.sync_copy(rowsb, acc_sh.at[v_stage], add=True)

    for seg in range(2):
        nseg = jnp.clip(nck - seg * K3SEG, 0, K3SEG)
        base_e = (start_chunk + seg * K3SEG) * K1C
        pltpu.sync_copy(u_h.at[pl.ds(base_e, K3SPAN)], u_all)
        pltpu.sync_copy(v_h.at[pl.ds(base_e, K3SPAN)], v_all)
        pltpu.sync_copy(ex_h.at[pl.ds(base_e, K3SPAN)], w_all)

        def wprep(g, carry):
            off = g * 16
            vseg = v_all[pl.ds(off, 16)]
            sv = plsc.load_gather(sv_t, [vseg])
            w_all[pl.ds(off, 16)] = w_all[pl.ds(off, 16)] / sv
            u_all[pl.ds(off, 16)] = u_all[pl.ds(off, 16)] + uoff
            return carry

        lax.fori_loop(0, nseg * (K1C // 16), wprep, 0)

        issue(0, rows0, sem0)

        def pair_body(p, carry):
            c0 = 2 * p
            wait(c0, rows0, sem0)

            @pl.when(c0 + 1 < nseg)
            def _():
                issue(c0 + 1, rows1, sem1)

            process(c0, rows0)

            @pl.when(c0 + 1 < nseg)
            def _():
                wait(c0 + 1, rows1, sem1)

                @pl.when(c0 + 2 < nseg)
                def _():
                    issue(c0 + 2, rows0, sem0)

                process(c0 + 1, rows1)
            return carry

        lax.fori_loop(0, (nseg + 1) // 2, pair_body, 0)

    plsc.subcore_barrier()
    pltpu.sync_copy(acc_sh.at[pl.ds(s * rows_per, rows_per)],
                    o_h.at[pl.ds(c * NT + s * rows_per, rows_per)])

    @pl.when(s == SC_NS - 1)
    def _out_tail():
        pltpu.sync_copy(acc_sh.at[pl.ds(SC_NS * rows_per, NT - SC_NS * rows_per)],
                        o_h.at[pl.ds(c * NT + SC_NS * rows_per, NT - SC_NS * rows_per)])


def _sc_scatter(ex, u_pad, v_pad, s_v, VqH, zeros):
    f = pl.kernel(
        _sc_scatter_body,
        out_type=[jax.ShapeDtypeStruct((SC_NC * NT, 128), jnp.float32)],
        mesh=plsc.VectorSubcoreMesh(**_MESH),
        compiler_params=pltpu.CompilerParams(needs_layout_passes=False),
        scratch_types=[
            pltpu.VMEM((K3SPAN,), jnp.int32),
            pltpu.VMEM((K3SPAN,), jnp.int32),
            pltpu.VMEM((K3SPAN,), jnp.float32),
            pltpu.VMEM((K1C,), jnp.int32),
            pltpu.VMEM((K1C, 128), jnp.float32),
            pltpu.VMEM((K1C, 128), jnp.float32),
            pltpu.VMEM((NT,), jnp.float32),
            pltpu.VMEM_SHARED((NT, 128), jnp.float32),
            pltpu.SemaphoreType.DMA,
            pltpu.SemaphoreType.DMA,
        ],
    )
    return f(ex, u_pad, v_pad, s_v, VqH, zeros)[0]


def kernel(Xq, Xt, u_idx, v_idx, Waq, baq, Wat, bat, Wvq, bvq, Wvt, bvt, Wmq, bmq, Wmt, bmt):
    Aq = _affine_elu(Xq, Waq, baq)
    At = _affine_elu(Xt, Wat, bat)
    Vt = _affine_elu(Xt, Wvt, bvt)
    VqH = _affine_elu_split(Xq, Wvq, bvq).reshape(2 * NQ, 128)

    u_pad = jnp.pad(u_idx, (0, EPAD - E))
    v_pad = jnp.pad(v_idx, (0, EPAD - E))

    logits, m_part, u_part = _sc_logits(Aq, At, u_pad, v_pad)
    m_v = _colreduce(m_part, "max")
    has_u = _colreduce(u_part, "max")

    ex, s_part = _sc_exp(logits, v_pad, m_v)
    s_v = _colreduce(s_part, "sum")

    zeros = jnp.zeros((NT, 128), jnp.float32)
    P = _sc_scatter(ex, u_pad, v_pad, s_v, VqH, zeros)

    Xt2q = Vt * has_u[:, None]
    Xq_merged = _merge2(Xq, Xt2q, Wmq, bmq)
    qrow = jnp.mean(Xq, axis=0) @ Wmt[2 * D:, :] + bmt
    Xt_merged = _merge_h(Xt, P, Wmt[:2 * D, :], qrow)
    return (Xq_merged, Xt_merged)


# speedup vs baseline: 13.4740x; 1.0801x over previous
import jax
import jax.numpy as jnp
from jax import lax
from jax.experimental import pallas as pl
from jax.experimental.pallas import tpu as pltpu
from jax.experimental.pallas import tpu_sc as plsc

NQ = 10000
NT = 10000
E = 160000
D = 256
BM = 1000

SC_NC = 2
SC_NS = 16
NW = SC_NC * SC_NS
CHUNK = 128
NCHUNKS = E // CHUNK
NGROUPS = CHUNK // 16

_MESH = dict(core_axis_name="c", subcore_axis_name="s", num_cores=SC_NC,
             num_subcores=SC_NS)


def _elu(x):
    return jnp.where(x > 0, x, jnp.exp(jnp.minimum(x, 0.0)) - 1.0)


def _affine_elu_body(x_ref, w_ref, b_ref, o_ref):
    acc = jnp.dot(x_ref[...], w_ref[...], preferred_element_type=jnp.float32)
    o_ref[...] = _elu(acc + b_ref[...])


def _affine_elu(x, w, b):
    n = x.shape[0]
    return pl.pallas_call(
        _affine_elu_body,
        grid=(n // BM,),
        in_specs=[
            pl.BlockSpec((BM, D), lambda i: (i, 0)),
            pl.BlockSpec((D, D), lambda i: (0, 0)),
            pl.BlockSpec((1, D), lambda i: (0, 0)),
        ],
        out_specs=pl.BlockSpec((BM, D), lambda i: (i, 0)),
        out_shape=jax.ShapeDtypeStruct((n, D), jnp.float32),
    )(x, w, b.reshape(1, D))


def _affine_elu_split_body(x_ref, w_ref, b_ref, o_ref):
    h = pl.program_id(1)
    acc = jnp.dot(x_ref[...], w_ref[:, pl.ds(h * 128, 128)],
                  preferred_element_type=jnp.float32)
    o_ref[...] = _elu(acc + b_ref[:, pl.ds(h * 128, 128)])[None]


def _affine_elu_split(x, w, b):
    n = x.shape[0]
    return pl.pallas_call(
        _affine_elu_split_body,
        grid=(n // BM, 2),
        in_specs=[
            pl.BlockSpec((BM, D), lambda i, h: (i, 0)),
            pl.BlockSpec((D, D), lambda i, h: (0, 0)),
            pl.BlockSpec((1, D), lambda i, h: (0, 0)),
        ],
        out_specs=pl.BlockSpec((1, BM, 128), lambda i, h: (h, i, 0)),
        out_shape=jax.ShapeDtypeStruct((2, n, 128), jnp.float32),
    )(x, w, b.reshape(1, D))


def _merge2_body(x_ref, t_ref, w_ref, b_ref, o_ref):
    acc = jnp.dot(x_ref[...], w_ref[:D, :], preferred_element_type=jnp.float32)
    acc += jnp.dot(t_ref[...], w_ref[D:, :], preferred_element_type=jnp.float32)
    o_ref[...] = acc + b_ref[...]


def _merge2(x, t, w, brow):
    n = x.shape[0]
    return pl.pallas_call(
        _merge2_body,
        grid=(n // BM,),
        in_specs=[
            pl.BlockSpec((BM, D), lambda i: (i, 0)),
            pl.BlockSpec((BM, D), lambda i: (i, 0)),
            pl.BlockSpec((2 * D, D), lambda i: (0, 0)),
            pl.BlockSpec((1, D), lambda i: (0, 0)),
        ],
        out_specs=pl.BlockSpec((BM, D), lambda i: (i, 0)),
        out_shape=jax.ShapeDtypeStruct((n, D), jnp.float32),
    )(x, t, w, brow.reshape(1, D))


def _merge_h_body(x_ref, h0_ref, h1_ref, w_ref, b_ref, o_ref):
    acc = jnp.dot(x_ref[...], w_ref[:D, :], preferred_element_type=jnp.float32)
    acc += jnp.dot(h0_ref[...], w_ref[D:D + 128, :],
                   preferred_element_type=jnp.float32)
    acc += jnp.dot(h1_ref[...], w_ref[D + 128:, :],
                   preferred_element_type=jnp.float32)
    o_ref[...] = acc + b_ref[...]


def _merge_h(x, halves, w, brow):
    n = x.shape[0]
    return pl.pallas_call(
        _merge_h_body,
        grid=(n // BM,),
        in_specs=[
            pl.BlockSpec((BM, D), lambda i: (i, 0)),
            pl.BlockSpec((BM, 128), lambda i: (i, 0)),
            pl.BlockSpec((BM, 128), lambda i: (i + n // BM, 0)),
            pl.BlockSpec((2 * D, D), lambda i: (0, 0)),
            pl.BlockSpec((1, D), lambda i: (0, 0)),
        ],
        out_specs=pl.BlockSpec((BM, D), lambda i: (i, 0)),
        out_shape=jax.ShapeDtypeStruct((n, D), jnp.float32),
    )(x, halves, halves, w, brow.reshape(1, D))


def _colreduce_body_max(x_ref, o_ref):
    o_ref[...] = jnp.max(x_ref[...], axis=0, keepdims=True)


def _colreduce_body_sum(x_ref, o_ref):
    o_ref[...] = jnp.sum(x_ref[...], axis=0, keepdims=True)


def _colreduce(x, kind):
    rows, n = x.shape
    body = _colreduce_body_max if kind == "max" else _colreduce_body_sum
    out = pl.pallas_call(
        body,
        grid=(1,),
        in_specs=[pl.BlockSpec((rows, n), lambda i: (0, 0))],
        out_specs=pl.BlockSpec((1, n), lambda i: (0, 0)),
        out_shape=jax.ShapeDtypeStruct((1, n), jnp.float32),
    )(x)
    return out.reshape(n)


def _wid(c, s):
    return s * SC_NC + c


def _nchunks_for(wid, stride):
    return (NCHUNKS - wid + stride - 1) // stride


K1C = 64
K1NCH = E // K1C
K1BASE = K1NCH // NW
K1SPAN = (K1BASE + 1) * K1C
EPAD = NW * K1SPAN


def _sc_logits_body(aq_h, at_h, u_h, v_h, lg_h, mp_h, up_h,
                    u_all, v_all, aq0, at0, aq1, at1, lg_all, tr_v,
                    m_loc, um_loc, sa0, sb0, sa1, sb1):
    c = lax.axis_index("c")
    s = lax.axis_index("s")
    wid = _wid(c, s)
    iota = lax.iota(jnp.int32, 16)
    neg = jnp.full((16,), -1e30, jnp.float32)
    one = jnp.ones((16,), jnp.float32)
    zero = jnp.zeros((16,), jnp.float32)

    def init_body(i, carry):
        m_loc[pl.ds(i * 16, 16)] = neg
        um_loc[pl.ds(i * 16, 16)] = zero
        return carry

    lax.fori_loop(0, NT // 16, init_body, 0)

    nck = K1BASE + jnp.where(wid < 4, 1, 0)
    start_chunk = wid * K1BASE + jnp.minimum(wid, 4)
    base_e = start_chunk * K1C
    pltpu.sync_copy(u_h.at[pl.ds(base_e, K1SPAN)], u_all)
    pltpu.sync_copy(v_h.at[pl.ds(base_e, K1SPAN)], v_all)

    def issue(ci, aqb, atb, sa, sb):
        ub = u_all.at[pl.ds(ci * K1C, K1C)]
        vb = v_all.at[pl.ds(ci * K1C, K1C)]
        pltpu.async_copy(aq_h.at[ub], aqb, sa)
        pltpu.async_copy(at_h.at[vb], atb, sb)

    def wait(ci, aqb, atb, sa, sb):
        ub = u_all.at[pl.ds(ci * K1C, K1C)]
        vb = v_all.at[pl.ds(ci * K1C, K1C)]
        pltpu.make_async_copy(aq_h.at[ub], aqb, sa).wait()
        pltpu.make_async_copy(at_h.at[vb], atb, sb).wait()

    def process(ci, aq_v, at_v):
        def group_body(g, carry2):
            gb = g * 16
            eb = ci * K1C + gb
            for j in range(16):
                acc = aq_v[gb + j, pl.ds(0, 16)] * at_v[gb + j, pl.ds(0, 16)]
                for k in range(1, 16):
                    acc = acc + (aq_v[gb + j, pl.ds(k * 16, 16)]
                                 * at_v[gb + j, pl.ds(k * 16, 16)])
                tr_v[pl.ds(j * 17, 16)] = acc
            lg = plsc.load_gather(tr_v, [iota * 17])
            for k in range(1, 16):
                lg = lg + plsc.load_gather(tr_v, [iota * 17 + k])
            lg_all[pl.ds(eb, 16)] = lg
            vseg = v_all[pl.ds(eb, 16)]
            useg = u_all[pl.ds(eb, 16)]
            plsc.store_scatter(um_loc, [useg], one)

            def mx_once(_):
                cur = plsc.load_gather(m_loc, [vseg])
                need = lg > cur
                plsc.store_scatter(m_loc, [vseg], jnp.maximum(cur, lg),
                                   mask=need)
                cur2 = plsc.load_gather(m_loc, [vseg])
                return jnp.any(lg > cur2)

            lax.while_loop(lambda p: p, mx_once, mx_once(True))
            return carry2

        lax.fori_loop(0, K1C // 16, group_body, 0)

    issue(0, aq0, at0, sa0, sb0)

    def pair_body(p, carry):
        c0 = 2 * p
        wait(c0, aq0, at0, sa0, sb0)
        issue(c0 + 1, aq1, at1, sa1, sb1)
        process(c0, aq0, at0)
        wait(c0 + 1, aq1, at1, sa1, sb1)

        @pl.when(c0 + 2 < nck)
        def _():
            issue(c0 + 2, aq0, at0, sa0, sb0)

        process(c0 + 1, aq1, at1)
        return carry

    lax.fori_loop(0, K1BASE // 2, pair_body, 0)

    @pl.when(nck == K1BASE + 1)
    def _last():
        wait(K1BASE, aq0, at0, sa0, sb0)
        process(K1BASE, aq0, at0)

    n_main = K1BASE * K1C
    pltpu.sync_copy(lg_all.at[pl.ds(0, n_main)], lg_h.at[pl.ds(base_e, n_main)])

    @pl.when(nck == K1BASE + 1)
    def _tail():
        pltpu.sync_copy(lg_all.at[pl.ds(n_main, K1C)],
                        lg_h.at[pl.ds(base_e + n_main, K1C)])

    pltpu.sync_copy(m_loc, mp_h.at[wid])
    pltpu.sync_copy(um_loc, up_h.at[wid])


def _sc_logits(Aq, At, u_pad, v_pad):
    f = pl.kernel(
        _sc_logits_body,
        out_type=[
            jax.ShapeDtypeStruct((E,), jnp.float32),
            jax.ShapeDtypeStruct((NW, NT), jnp.float32),
            jax.ShapeDtypeStruct((NW, NQ), jnp.float32),
        ],
        mesh=plsc.VectorSubcoreMesh(**_MESH),
        compiler_params=pltpu.CompilerParams(needs_layout_passes=False),
        scratch_types=[
            pltpu.VMEM((K1SPAN,), jnp.int32),
            pltpu.VMEM((K1SPAN,), jnp.int32),
            pltpu.VMEM((K1C, D), jnp.float32),
            pltpu.VMEM((K1C, D), jnp.float32),
            pltpu.VMEM((K1C, D), jnp.float32),
            pltpu.VMEM((K1C, D), jnp.float32),
            pltpu.VMEM((K1SPAN,), jnp.float32),
            pltpu.VMEM((16 * 17,), jnp.float32),
            pltpu.VMEM((NT,), jnp.float32),
            pltpu.VMEM((NQ,), jnp.float32),
            pltpu.SemaphoreType.DMA,
            pltpu.SemaphoreType.DMA,
            pltpu.SemaphoreType.DMA,
            pltpu.SemaphoreType.DMA,
        ],
    )
    return f(Aq, At, u_pad, v_pad)


def _sc_exp_body(lg_h, v_h, mv_h, ex_h, sp_h,
                 v_all, lg_all, mv_t, s_loc, tag, sem):
    c = lax.axis_index("c")
    s = lax.axis_index("s")
    wid = _wid(c, s)
    iota = lax.iota(jnp.int32, 16)
    zero = jnp.zeros((16,), jnp.float32)

    pltpu.sync_copy(mv_h, mv_t)

    def init_body(i, carry):
        s_loc[pl.ds(i * 16, 16)] = zero
        return carry

    lax.fori_loop(0, NT // 16, init_body, 0)

    nck = K1BASE + jnp.where(wid < 4, 1, 0)
    start_chunk = wid * K1BASE + jnp.minimum(wid, 4)
    base_e = start_chunk * K1C
    pltpu.sync_copy(v_h.at[pl.ds(base_e, K1SPAN)], v_all)
    pltpu.sync_copy(lg_h.at[pl.ds(base_e, K1SPAN)], lg_all)

    def group_body(g, carry2):
        gb = g * 16
        vseg = v_all[pl.ds(gb, 16)]
        m = plsc.load_gather(mv_t, [vseg])
        ex = jnp.exp(lg_all[pl.ds(gb, 16)] - m)
        lg_all[pl.ds(gb, 16)] = ex

        def claim(st):
            plsc.store_scatter(tag, [vseg], iota, mask=st)
            t = plsc.load_gather(tag, [vseg])
            win = st & (t == iota)
            cur = plsc.load_gather(s_loc, [vseg])
            plsc.store_scatter(s_loc, [vseg], cur + ex, mask=win)
            return st & jnp.logical_not(win)

        st0 = iota == iota
        lax.while_loop(lambda st: jnp.any(st), claim, claim(st0))
        return carry2

    lax.fori_loop(0, nck * (K1C // 16), group_body, 0)

    n_main = K1BASE * K1C
    pltpu.sync_copy(lg_all.at[pl.ds(0, n_main)], ex_h.at[pl.ds(base_e, n_main)])

    @pl.when(nck == K1BASE + 1)
    def _tail():
        pltpu.sync_copy(lg_all.at[pl.ds(n_main, K1C)],
                        ex_h.at[pl.ds(base_e + n_main, K1C)])

    pltpu.sync_copy(s_loc, sp_h.at[wid])


def _sc_exp(logits, v_idx, m_v):
    f = pl.kernel(
        _sc_exp_body,
        out_type=[
            jax.ShapeDtypeStruct((EPAD,), jnp.float32),
            jax.ShapeDtypeStruct((NW, NT), jnp.float32),
        ],
        mesh=plsc.VectorSubcoreMesh(**_MESH),
        compiler_params=pltpu.CompilerParams(needs_layout_passes=False),
        scratch_types=[
            pltpu.VMEM((K1SPAN,), jnp.int32),
            pltpu.VMEM((K1SPAN,), jnp.float32),
            pltpu.VMEM((NT,), jnp.float32),
            pltpu.VMEM((NT,), jnp.float32),
            pltpu.VMEM((NT,), jnp.int32),
            pltpu.SemaphoreType.DMA,
        ],
    )
    return f(logits, v_idx, m_v)


K3BASE = K1NCH // SC_NS
K3SEG = (K1BASE + 2) // 2 * 2
K3SPAN = K3SEG * K1C


def _sc_scatter_body(ex_h, u_h, v_h, sv_h, vqh_h, zero_h, o_h,
                     u_all, v_all, w_all, v_st0, v_st1, rows0, rows1,
                     sv_t, acc_sh, sem0, sem1, sem_s):
    c = lax.axis_index("c")
    s = lax.axis_index("s")
    rows_per = 624
    uoff = c * NQ

    pltpu.sync_copy(sv_h, sv_t)
    pltpu.sync_copy(zero_h.at[pl.ds(s * rows_per, rows_per)],
                    acc_sh.at[pl.ds(s * rows_per, rows_per)])

    @pl.when(s == SC_NS - 1)
    def _zero_tail():
        pltpu.sync_copy(zero_h.at[pl.ds(SC_NS * rows_per, NT - SC_NS * rows_per)],
                        acc_sh.at[pl.ds(SC_NS * rows_per, NT - SC_NS * rows_per)])

    nck = K3BASE + jnp.where(s < 4, 1, 0)
    start_chunk = s * K3BASE + jnp.minimum(s, 4)
    plsc.subcore_barrier()

    def issue(ci, rowsb, sem):
        ub = u_all.at[pl.ds(ci * K1C, K1C)]
        pltpu.async_copy(vqh_h.at[ub], rowsb, sem)

    def wait(ci, rowsb, sem):
        ub = u_all.at[pl.ds(ci * K1C, K1C)]
        pltpu.make_async_copy(vqh_h.at[ub], rowsb, sem).wait()

    H = K1C // 2

    def process(ci, rowsb):
        def row_body(j, carry2):
            wrow = plsc.load_gather(
                w_all, [jnp.full((16,), ci * K1C + j, jnp.int32)])
            for k in range(8):
                rowsb[j, pl.ds(k * 16, 16)] = (
                    rowsb[j, pl.ds(k * 16, 16)] * wrow)
            return carry2

        for t in range(H // 16):
            v_st0[pl.ds(t * 16, 16)] = v_all[pl.ds(ci * K1C + t * 16, 16)]
            v_st1[pl.ds(t * 16, 16)] = v_all[pl.ds(ci * K1C + H + t * 16, 16)]
        lax.fori_loop(0, H, row_body, 0, unroll=4)
        pltpu.async_copy(rowsb.at[pl.ds(0, H)], acc_sh.at[v_st0], sem_s,
                         add=True)
        lax.fori_loop(H, K1C, row_body, 0, unroll=4)
        pltpu.async_copy(rowsb.at[pl.ds(H, H)], acc_sh.at[v_st1], sem_s,
                         add=True)
        pltpu.make_async_copy(rowsb.at[pl.ds(0, H)], acc_sh.at[v_st0],
                              sem_s).wait()
        pltpu.make_async_copy(rowsb.at[pl.ds(H, H)], acc_sh.at[v_st1],
                              sem_s).wait()

    for seg in range(2):
        nseg = jnp.clip(nck - seg * K3SEG, 0, K3SEG)
        base_e = (start_chunk + seg * K3SEG) * K1C
        pltpu.sync_copy(u_h.at[pl.ds(base_e, K3SPAN)], u_all)
        pltpu.sync_copy(v_h.at[pl.ds(base_e, K3SPAN)], v_all)
        pltpu.sync_copy(ex_h.at[pl.ds(base_e, K3SPAN)], w_all)

        def wprep(g, carry):
            off = g * 16
            vseg = v_all[pl.ds(off, 16)]
            sv = plsc.load_gather(sv_t, [vseg])
            w_all[pl.ds(off, 16)] = w_all[pl.ds(off, 16)] / sv
            u_all[pl.ds(off, 16)] = u_all[pl.ds(off, 16)] + uoff
            return carry

        lax.fori_loop(0, nseg * (K1C // 16), wprep, 0)

        issue(0, rows0, sem0)

        def pair_body(p, carry):
            c0 = 2 * p
            wait(c0, rows0, sem0)

            @pl.when(c0 + 1 < nseg)
            def _():
                issue(c0 + 1, rows1, sem1)

            process(c0, rows0)

            @pl.when(c0 + 1 < nseg)
            def _():
                wait(c0 + 1, rows1, sem1)

                @pl.when(c0 + 2 < nseg)
                def _():
                    issue(c0 + 2, rows0, sem0)

                process(c0 + 1, rows1)
            return carry

        lax.fori_loop(0, (nseg + 1) // 2, pair_body, 0)

    plsc.subcore_barrier()
    pltpu.sync_copy(acc_sh.at[pl.ds(s * rows_per, rows_per)],
                    o_h.at[pl.ds(c * NT + s * rows_per, rows_per)])

    @pl.when(s == SC_NS - 1)
    def _out_tail():
        pltpu.sync_copy(acc_sh.at[pl.ds(SC_NS * rows_per, NT - SC_NS * rows_per)],
                        o_h.at[pl.ds(c * NT + SC_NS * rows_per, NT - SC_NS * rows_per)])


def _sc_scatter(ex, u_pad, v_pad, s_v, VqH, zeros):
    f = pl.kernel(
        _sc_scatter_body,
        out_type=[jax.ShapeDtypeStruct((SC_NC * NT, 128), jnp.float32)],
        mesh=plsc.VectorSubcoreMesh(**_MESH),
        compiler_params=pltpu.CompilerParams(needs_layout_passes=False),
        scratch_types=[
            pltpu.VMEM((K3SPAN,), jnp.int32),
            pltpu.VMEM((K3SPAN,), jnp.int32),
            pltpu.VMEM((K3SPAN,), jnp.float32),
            pltpu.VMEM((K1C // 2,), jnp.int32),
            pltpu.VMEM((K1C // 2,), jnp.int32),
            pltpu.VMEM((K1C, 128), jnp.float32),
            pltpu.VMEM((K1C, 128), jnp.float32),
            pltpu.VMEM((NT,), jnp.float32),
            pltpu.VMEM_SHARED((NT, 128), jnp.float32),
            pltpu.SemaphoreType.DMA,
            pltpu.SemaphoreType.DMA,
            pltpu.SemaphoreType.DMA,
        ],
    )
    return f(ex, u_pad, v_pad, s_v, VqH, zeros)[0]


def kernel(Xq, Xt, u_idx, v_idx, Waq, baq, Wat, bat, Wvq, bvq, Wvt, bvt, Wmq, bmq, Wmt, bmt):
    Aq = _affine_elu(Xq, Waq, baq)
    At = _affine_elu(Xt, Wat, bat)
    Vt = _affine_elu(Xt, Wvt, bvt)
    VqH = _affine_elu_split(Xq, Wvq, bvq).reshape(2 * NQ, 128)

    u_pad = jnp.pad(u_idx, (0, EPAD - E))
    v_pad = jnp.pad(v_idx, (0, EPAD - E))

    logits, m_part, u_part = _sc_logits(Aq, At, u_pad, v_pad)
    m_v = _colreduce(m_part, "max")
    has_u = _colreduce(u_part, "max")

    ex, s_part = _sc_exp(logits, v_pad, m_v)
    s_v = _colreduce(s_part, "sum")

    zeros = jnp.zeros((NT, 128), jnp.float32)
    P = _sc_scatter(ex, u_pad, v_pad, s_v, VqH, zeros)

    Xt2q = Vt * has_u[:, None]
    Xq_merged = _merge2(Xq, Xt2q, Wmq, bmq)
    qrow = jnp.mean(Xq, axis=0) @ Wmt[2 * D:, :] + bmt
    Xt_merged = _merge_h(Xt, P, Wmt[:2 * D, :], qrow)
    return (Xq_merged, Xt_merged)


# speedup vs baseline: 13.5092x; 1.0026x over previous
import jax
import jax.numpy as jnp
from jax import lax
from jax.experimental import pallas as pl
from jax.experimental.pallas import tpu as pltpu
from jax.experimental.pallas import tpu_sc as plsc

NQ = 10000
NT = 10000
E = 160000
D = 256
BM = 1000

SC_NC = 2
SC_NS = 16
NW = SC_NC * SC_NS
CHUNK = 128
NCHUNKS = E // CHUNK
NGROUPS = CHUNK // 16

_MESH = dict(core_axis_name="c", subcore_axis_name="s", num_cores=SC_NC,
             num_subcores=SC_NS)


def _elu(x):
    return jnp.where(x > 0, x, jnp.exp(jnp.minimum(x, 0.0)) - 1.0)


def _affine_elu_body(x_ref, w_ref, b_ref, o_ref):
    acc = jnp.dot(x_ref[...], w_ref[...], preferred_element_type=jnp.float32)
    o_ref[...] = _elu(acc + b_ref[...])


def _affine_elu(x, w, b):
    n = x.shape[0]
    return pl.pallas_call(
        _affine_elu_body,
        grid=(n // BM,),
        in_specs=[
            pl.BlockSpec((BM, D), lambda i: (i, 0)),
            pl.BlockSpec((D, D), lambda i: (0, 0)),
            pl.BlockSpec((1, D), lambda i: (0, 0)),
        ],
        out_specs=pl.BlockSpec((BM, D), lambda i: (i, 0)),
        out_shape=jax.ShapeDtypeStruct((n, D), jnp.float32),
    )(x, w, b.reshape(1, D))


def _affine_elu_split_body(x_ref, w_ref, b_ref, o_ref):
    h = pl.program_id(1)
    acc = jnp.dot(x_ref[...], w_ref[:, pl.ds(h * 128, 128)],
                  preferred_element_type=jnp.float32)
    o_ref[...] = _elu(acc + b_ref[:, pl.ds(h * 128, 128)])[None]


def _affine_elu_split(x, w, b):
    n = x.shape[0]
    return pl.pallas_call(
        _affine_elu_split_body,
        grid=(n // BM, 2),
        in_specs=[
            pl.BlockSpec((BM, D), lambda i, h: (i, 0)),
            pl.BlockSpec((D, D), lambda i, h: (0, 0)),
            pl.BlockSpec((1, D), lambda i, h: (0, 0)),
        ],
        out_specs=pl.BlockSpec((1, BM, 128), lambda i, h: (h, i, 0)),
        out_shape=jax.ShapeDtypeStruct((2, n, 128), jnp.float32),
    )(x, w, b.reshape(1, D))


def _merge2_body(x_ref, t_ref, w_ref, b_ref, o_ref):
    acc = jnp.dot(x_ref[...], w_ref[:D, :], preferred_element_type=jnp.float32)
    acc += jnp.dot(t_ref[...], w_ref[D:, :], preferred_element_type=jnp.float32)
    o_ref[...] = acc + b_ref[...]


def _merge2(x, t, w, brow):
    n = x.shape[0]
    return pl.pallas_call(
        _merge2_body,
        grid=(n // BM,),
        in_specs=[
            pl.BlockSpec((BM, D), lambda i: (i, 0)),
            pl.BlockSpec((BM, D), lambda i: (i, 0)),
            pl.BlockSpec((2 * D, D), lambda i: (0, 0)),
            pl.BlockSpec((1, D), lambda i: (0, 0)),
        ],
        out_specs=pl.BlockSpec((BM, D), lambda i: (i, 0)),
        out_shape=jax.ShapeDtypeStruct((n, D), jnp.float32),
    )(x, t, w, brow.reshape(1, D))


def _merge_h_body(x_ref, h0_ref, h1_ref, w_ref, b_ref, o_ref):
    acc = jnp.dot(x_ref[...], w_ref[:D, :], preferred_element_type=jnp.float32)
    acc += jnp.dot(h0_ref[...], w_ref[D:D + 128, :],
                   preferred_element_type=jnp.float32)
    acc += jnp.dot(h1_ref[...], w_ref[D + 128:, :],
                   preferred_element_type=jnp.float32)
    o_ref[...] = acc + b_ref[...]


def _merge_h(x, halves, w, brow):
    n = x.shape[0]
    return pl.pallas_call(
        _merge_h_body,
        grid=(n // BM,),
        in_specs=[
            pl.BlockSpec((BM, D), lambda i: (i, 0)),
            pl.BlockSpec((BM, 128), lambda i: (i, 0)),
            pl.BlockSpec((BM, 128), lambda i: (i + n // BM, 0)),
            pl.BlockSpec((2 * D, D), lambda i: (0, 0)),
            pl.BlockSpec((1, D), lambda i: (0, 0)),
        ],
        out_specs=pl.BlockSpec((BM, D), lambda i: (i, 0)),
        out_shape=jax.ShapeDtypeStruct((n, D), jnp.float32),
    )(x, halves, halves, w, brow.reshape(1, D))


def _colreduce_body_max(x_ref, o_ref):
    o_ref[...] = jnp.max(x_ref[...], axis=0, keepdims=True)


def _colreduce_body_sum(x_ref, o_ref):
    o_ref[...] = jnp.sum(x_ref[...], axis=0, keepdims=True)


def _colreduce(x, kind):
    rows, n = x.shape
    body = _colreduce_body_max if kind == "max" else _colreduce_body_sum
    out = pl.pallas_call(
        body,
        grid=(1,),
        in_specs=[pl.BlockSpec((rows, n), lambda i: (0, 0))],
        out_specs=pl.BlockSpec((1, n), lambda i: (0, 0)),
        out_shape=jax.ShapeDtypeStruct((1, n), jnp.float32),
    )(x)
    return out.reshape(n)


def _wid(c, s):
    return s * SC_NC + c


def _nchunks_for(wid, stride):
    return (NCHUNKS - wid + stride - 1) // stride


K1C = 64
K1NCH = E // K1C
K1BASE = K1NCH // NW
K1SPAN = (K1BASE + 1) * K1C
EPAD = NW * K1SPAN


def _sc_logits_body(aq_h, at_h, u_h, v_h, lg_h, mp_h, up_h,
                    u_all, v_all, aq0, at0, aq1, at1, lg_all, tr_v,
                    m_loc, um_loc, sa0, sb0, sa1, sb1):
    c = lax.axis_index("c")
    s = lax.axis_index("s")
    wid = _wid(c, s)
    iota = lax.iota(jnp.int32, 16)
    neg = jnp.full((16,), -1e30, jnp.float32)
    one = jnp.ones((16,), jnp.float32)
    zero = jnp.zeros((16,), jnp.float32)

    def init_body(i, carry):
        m_loc[pl.ds(i * 16, 16)] = neg
        um_loc[pl.ds(i * 16, 16)] = zero
        return carry

    lax.fori_loop(0, NT // 16, init_body, 0)

    nck = K1BASE + jnp.where(wid < 4, 1, 0)
    start_chunk = wid * K1BASE + jnp.minimum(wid, 4)
    base_e = start_chunk * K1C
    pltpu.sync_copy(u_h.at[pl.ds(base_e, K1SPAN)], u_all)
    pltpu.sync_copy(v_h.at[pl.ds(base_e, K1SPAN)], v_all)

    def issue(ci, aqb, atb, sa, sb):
        ub = u_all.at[pl.ds(ci * K1C, K1C)]
        vb = v_all.at[pl.ds(ci * K1C, K1C)]
        pltpu.async_copy(aq_h.at[ub], aqb, sa)
        pltpu.async_copy(at_h.at[vb], atb, sb)

    def wait(ci, aqb, atb, sa, sb):
        ub = u_all.at[pl.ds(ci * K1C, K1C)]
        vb = v_all.at[pl.ds(ci * K1C, K1C)]
        pltpu.make_async_copy(aq_h.at[ub], aqb, sa).wait()
        pltpu.make_async_copy(at_h.at[vb], atb, sb).wait()

    def process(ci, aq_v, at_v):
        def group_body(g, carry2):
            gb = g * 16
            eb = ci * K1C + gb
            for j in range(16):
                acc = aq_v[gb + j, pl.ds(0, 16)] * at_v[gb + j, pl.ds(0, 16)]
                for k in range(1, 16):
                    acc = acc + (aq_v[gb + j, pl.ds(k * 16, 16)]
                                 * at_v[gb + j, pl.ds(k * 16, 16)])
                tr_v[pl.ds(j * 17, 16)] = acc
            lg = plsc.load_gather(tr_v, [iota * 17])
            for k in range(1, 16):
                lg = lg + plsc.load_gather(tr_v, [iota * 17 + k])
            lg_all[pl.ds(eb, 16)] = lg
            vseg = v_all[pl.ds(eb, 16)]
            useg = u_all[pl.ds(eb, 16)]
            plsc.store_scatter(um_loc, [useg], one)

            def mx_once(_):
                cur = plsc.load_gather(m_loc, [vseg])
                need = lg > cur
                plsc.store_scatter(m_loc, [vseg], jnp.maximum(cur, lg),
                                   mask=need)
                cur2 = plsc.load_gather(m_loc, [vseg])
                return jnp.any(lg > cur2)

            lax.while_loop(lambda p: p, mx_once, mx_once(True))
            return carry2

        lax.fori_loop(0, K1C // 16, group_body, 0)

    issue(0, aq0, at0, sa0, sb0)

    def pair_body(p, carry):
        c0 = 2 * p
        wait(c0, aq0, at0, sa0, sb0)
        issue(c0 + 1, aq1, at1, sa1, sb1)
        process(c0, aq0, at0)
        wait(c0 + 1, aq1, at1, sa1, sb1)

        @pl.when(c0 + 2 < nck)
        def _():
            issue(c0 + 2, aq0, at0, sa0, sb0)

        process(c0 + 1, aq1, at1)
        return carry

    lax.fori_loop(0, K1BASE // 2, pair_body, 0)

    @pl.when(nck == K1BASE + 1)
    def _last():
        wait(K1BASE, aq0, at0, sa0, sb0)
        process(K1BASE, aq0, at0)

    n_main = K1BASE * K1C
    pltpu.sync_copy(lg_all.at[pl.ds(0, n_main)], lg_h.at[pl.ds(base_e, n_main)])

    @pl.when(nck == K1BASE + 1)
    def _tail():
        pltpu.sync_copy(lg_all.at[pl.ds(n_main, K1C)],
                        lg_h.at[pl.ds(base_e + n_main, K1C)])

    pltpu.sync_copy(m_loc, mp_h.at[wid])
    pltpu.sync_copy(um_loc, up_h.at[wid])


def _sc_logits(Aq, At, u_pad, v_pad):
    f = pl.kernel(
        _sc_logits_body,
        out_type=[
            jax.ShapeDtypeStruct((E,), jnp.float32),
            jax.ShapeDtypeStruct((NW, NT), jnp.float32),
            jax.ShapeDtypeStruct((NW, NQ), jnp.float32),
        ],
        mesh=plsc.VectorSubcoreMesh(**_MESH),
        compiler_params=pltpu.CompilerParams(needs_layout_passes=False),
        scratch_types=[
            pltpu.VMEM((K1SPAN,), jnp.int32),
            pltpu.VMEM((K1SPAN,), jnp.int32),
            pltpu.VMEM((K1C, D), jnp.float32),
            pltpu.VMEM((K1C, D), jnp.float32),
            pltpu.VMEM((K1C, D), jnp.float32),
            pltpu.VMEM((K1C, D), jnp.float32),
            pltpu.VMEM((K1SPAN,), jnp.float32),
            pltpu.VMEM((16 * 17,), jnp.float32),
            pltpu.VMEM((NT,), jnp.float32),
            pltpu.VMEM((NQ,), jnp.float32),
            pltpu.SemaphoreType.DMA,
            pltpu.SemaphoreType.DMA,
            pltpu.SemaphoreType.DMA,
            pltpu.SemaphoreType.DMA,
        ],
    )
    return f(Aq, At, u_pad, v_pad)


def _sc_exp_body(lg_h, v_h, mv_h, ex_h, sp_h,
                 v_all, lg_all, mv_t, s_loc, tag, sem):
    c = lax.axis_index("c")
    s = lax.axis_index("s")
    wid = _wid(c, s)
    iota = lax.iota(jnp.int32, 16)
    zero = jnp.zeros((16,), jnp.float32)

    pltpu.sync_copy(mv_h, mv_t)

    def init_body(i, carry):
        s_loc[pl.ds(i * 16, 16)] = zero
        return carry

    lax.fori_loop(0, NT // 16, init_body, 0)

    nck = K1BASE + jnp.where(wid < 4, 1, 0)
    start_chunk = wid * K1BASE + jnp.minimum(wid, 4)
    base_e = start_chunk * K1C
    pltpu.sync_copy(v_h.at[pl.ds(base_e, K1SPAN)], v_all)
    pltpu.sync_copy(lg_h.at[pl.ds(base_e, K1SPAN)], lg_all)

    def group_body(g, carry2):
        gb = g * 16
        vseg = v_all[pl.ds(gb, 16)]
        m = plsc.load_gather(mv_t, [vseg])
        ex = jnp.exp(lg_all[pl.ds(gb, 16)] - m)
        lg_all[pl.ds(gb, 16)] = ex

        def claim(st):
            plsc.store_scatter(tag, [vseg], iota, mask=st)
            t = plsc.load_gather(tag, [vseg])
            win = st & (t == iota)
            cur = plsc.load_gather(s_loc, [vseg])
            plsc.store_scatter(s_loc, [vseg], cur + ex, mask=win)
            return st & jnp.logical_not(win)

        st0 = iota == iota
        lax.while_loop(lambda st: jnp.any(st), claim, claim(st0))
        return carry2

    lax.fori_loop(0, nck * (K1C // 16), group_body, 0)

    n_main = K1BASE * K1C
    pltpu.sync_copy(lg_all.at[pl.ds(0, n_main)], ex_h.at[pl.ds(base_e, n_main)])

    @pl.when(nck == K1BASE + 1)
    def _tail():
        pltpu.sync_copy(lg_all.at[pl.ds(n_main, K1C)],
                        ex_h.at[pl.ds(base_e + n_main, K1C)])

    pltpu.sync_copy(s_loc, sp_h.at[wid])


def _sc_exp(logits, v_idx, m_v):
    f = pl.kernel(
        _sc_exp_body,
        out_type=[
            jax.ShapeDtypeStruct((EPAD,), jnp.float32),
            jax.ShapeDtypeStruct((NW, NT), jnp.float32),
        ],
        mesh=plsc.VectorSubcoreMesh(**_MESH),
        compiler_params=pltpu.CompilerParams(needs_layout_passes=False),
        scratch_types=[
            pltpu.VMEM((K1SPAN,), jnp.int32),
            pltpu.VMEM((K1SPAN,), jnp.float32),
            pltpu.VMEM((NT,), jnp.float32),
            pltpu.VMEM((NT,), jnp.float32),
            pltpu.VMEM((NT,), jnp.int32),
            pltpu.SemaphoreType.DMA,
        ],
    )
    return f(logits, v_idx, m_v)


K3BASE = K1NCH // SC_NS
K3SEG = (K1BASE + 2) // 2 * 2
K3SPAN = K3SEG * K1C


def _sc_scatter_body(ex_h, u_h, v_h, sv_h, vqh_h, zero_h, o_h,
                     u_all, v_all, w_all, v_st0, v_st1, rows0, rows1,
                     sv_t, acc_sh, sem0, sem1, sem_s):
    c = lax.axis_index("c")
    s = lax.axis_index("s")
    rows_per = 624
    uoff = c * NQ

    pltpu.sync_copy(sv_h, sv_t)
    pltpu.sync_copy(zero_h.at[pl.ds(s * rows_per, rows_per)],
                    acc_sh.at[pl.ds(s * rows_per, rows_per)])

    @pl.when(s == SC_NS - 1)
    def _zero_tail():
        pltpu.sync_copy(zero_h.at[pl.ds(SC_NS * rows_per, NT - SC_NS * rows_per)],
                        acc_sh.at[pl.ds(SC_NS * rows_per, NT - SC_NS * rows_per)])

    nck = K3BASE + jnp.where(s < 4, 1, 0)
    start_chunk = s * K3BASE + jnp.minimum(s, 4)
    plsc.subcore_barrier()

    def issue(ci, rowsb, sem):
        ub = u_all.at[pl.ds(ci * K1C, K1C)]
        pltpu.async_copy(vqh_h.at[ub], rowsb, sem)

    def wait(ci, rowsb, sem):
        ub = u_all.at[pl.ds(ci * K1C, K1C)]
        pltpu.make_async_copy(vqh_h.at[ub], rowsb, sem).wait()

    H = K1C // 2

    def process(ci, rowsb):
        def grp_body(gq, carry2):
            cb = gq * 16
            wvec = w_all[pl.ds(ci * K1C + cb, 16)]
            for r in range(16):
                wrow = wvec.at[jnp.full((16,), r, jnp.int32)].get(
                    mode="promise_in_bounds")
                for k in range(8):
                    rowsb[cb + r, pl.ds(k * 16, 16)] = (
                        rowsb[cb + r, pl.ds(k * 16, 16)] * wrow)
            return carry2

        for t in range(H // 16):
            v_st0[pl.ds(t * 16, 16)] = v_all[pl.ds(ci * K1C + t * 16, 16)]
            v_st1[pl.ds(t * 16, 16)] = v_all[pl.ds(ci * K1C + H + t * 16, 16)]
        lax.fori_loop(0, H // 16, grp_body, 0)
        pltpu.async_copy(rowsb.at[pl.ds(0, H)], acc_sh.at[v_st0], sem_s,
                         add=True)
        lax.fori_loop(H // 16, K1C // 16, grp_body, 0)
        pltpu.async_copy(rowsb.at[pl.ds(H, H)], acc_sh.at[v_st1], sem_s,
                         add=True)
        pltpu.make_async_copy(rowsb.at[pl.ds(0, H)], acc_sh.at[v_st0],
                              sem_s).wait()
        pltpu.make_async_copy(rowsb.at[pl.ds(H, H)], acc_sh.at[v_st1],
                              sem_s).wait()

    for seg in range(2):
        nseg = jnp.clip(nck - seg * K3SEG, 0, K3SEG)
        base_e = (start_chunk + seg * K3SEG) * K1C
        pltpu.sync_copy(u_h.at[pl.ds(base_e, K3SPAN)], u_all)
        pltpu.sync_copy(v_h.at[pl.ds(base_e, K3SPAN)], v_all)
        pltpu.sync_copy(ex_h.at[pl.ds(base_e, K3SPAN)], w_all)

        def wprep(g, carry):
            off = g * 16
            vseg = v_all[pl.ds(off, 16)]
            sv = plsc.load_gather(sv_t, [vseg])
            w_all[pl.ds(off, 16)] = w_all[pl.ds(off, 16)] / sv
            u_all[pl.ds(off, 16)] = u_all[pl.ds(off, 16)] + uoff
            return carry

        lax.fori_loop(0, nseg * (K1C // 16), wprep, 0)

        issue(0, rows0, sem0)

        def pair_body(p, carry):
            c0 = 2 * p
            wait(c0, rows0, sem0)

            @pl.when(c0 + 1 < nseg)
            def _():
                issue(c0 + 1, rows1, sem1)

            process(c0, rows0)

            @pl.when(c0 + 1 < nseg)
            def _():
                wait(c0 + 1, rows1, sem1)

                @pl.when(c0 + 2 < nseg)
                def _():
                    issue(c0 + 2, rows0, sem0)

                process(c0 + 1, rows1)
            return carry

        lax.fori_loop(0, (nseg + 1) // 2, pair_body, 0)

    plsc.subcore_barrier()
    pltpu.sync_copy(acc_sh.at[pl.ds(s * rows_per, rows_per)],
                    o_h.at[pl.ds(c * NT + s * rows_per, rows_per)])

    @pl.when(s == SC_NS - 1)
    def _out_tail():
        pltpu.sync_copy(acc_sh.at[pl.ds(SC_NS * rows_per, NT - SC_NS * rows_per)],
                        o_h.at[pl.ds(c * NT + SC_NS * rows_per, NT - SC_NS * rows_per)])


def _sc_scatter(ex, u_pad, v_pad, s_v, VqH, zeros):
    f = pl.kernel(
        _sc_scatter_body,
        out_type=[jax.ShapeDtypeStruct((SC_NC * NT, 128), jnp.float32)],
        mesh=plsc.VectorSubcoreMesh(**_MESH),
        compiler_params=pltpu.CompilerParams(needs_layout_passes=False),
        scratch_types=[
            pltpu.VMEM((K3SPAN,), jnp.int32),
            pltpu.VMEM((K3SPAN,), jnp.int32),
            pltpu.VMEM((K3SPAN,), jnp.float32),
            pltpu.VMEM((K1C // 2,), jnp.int32),
            pltpu.VMEM((K1C // 2,), jnp.int32),
            pltpu.VMEM((K1C, 128), jnp.float32),
            pltpu.VMEM((K1C, 128), jnp.float32),
            pltpu.VMEM((NT,), jnp.float32),
            pltpu.VMEM_SHARED((NT, 128), jnp.float32),
            pltpu.SemaphoreType.DMA,
            pltpu.SemaphoreType.DMA,
            pltpu.SemaphoreType.DMA,
        ],
    )
    return f(ex, u_pad, v_pad, s_v, VqH, zeros)[0]


def kernel(Xq, Xt, u_idx, v_idx, Waq, baq, Wat, bat, Wvq, bvq, Wvt, bvt, Wmq, bmq, Wmt, bmt):
    Aq = _affine_elu(Xq, Waq, baq)
    At = _affine_elu(Xt, Wat, bat)
    Vt = _affine_elu(Xt, Wvt, bvt)
    VqH = _affine_elu_split(Xq, Wvq, bvq).reshape(2 * NQ, 128)

    u_pad = jnp.pad(u_idx, (0, EPAD - E))
    v_pad = jnp.pad(v_idx, (0, EPAD - E))

    logits, m_part, u_part = _sc_logits(Aq, At, u_pad, v_pad)
    m_v = _colreduce(m_part, "max")
    has_u = _colreduce(u_part, "max")

    ex, s_part = _sc_exp(logits, v_pad, m_v)
    s_v = _colreduce(s_part, "sum")

    zeros = jnp.zeros((NT, 128), jnp.float32)
    P = _sc_scatter(ex, u_pad, v_pad, s_v, VqH, zeros)

    Xt2q = Vt * has_u[:, None]
    Xq_merged = _merge2(Xq, Xt2q, Wmq, bmq)
    qrow = jnp.mean(Xq, axis=0) @ Wmt[2 * D:, :] + bmt
    Xt_merged = _merge_h(Xt, P, Wmt[:2 * D, :], qrow)
    return (Xq_merged, Xt_merged)
